# Initial kernel scaffold; baseline (speedup 1.0000x reference)
import jax
import jax.numpy as jnp
from jax.experimental import pallas as pl


def kernel(x, W, b):
    raise NotImplementedError("write your pallas kernel here")


# speedup vs baseline: 1.2604x; 1.2604x over previous
import functools

import jax
import jax.numpy as jnp
from jax import lax
from jax.experimental import pallas as pl
from jax.experimental.pallas import tpu as pltpu

_TOK_BLOCK = 512


def _routing_body(x_ref, wt_ref, b_ref, w1_ref, w2_ref, i1_ref, i2_ref,
                  dl_ref, acc_ref, *, n_tokens, n_experts):
    g = pl.program_id(0)
    ng = pl.num_programs(0)

    logits = jnp.dot(x_ref[...], wt_ref[...],
                     preferred_element_type=jnp.float32) + b_ref[...]

    t = logits.shape[0]
    iota = lax.broadcasted_iota(jnp.int32, (t, n_experts), 1)

    m1 = jnp.max(logits, axis=-1, keepdims=True)
    i1 = jnp.min(jnp.where(logits == m1, iota, n_experts), axis=-1,
                 keepdims=True)
    masked = jnp.where(iota == i1, -jnp.inf, logits)
    m2 = jnp.max(masked, axis=-1, keepdims=True)
    i2 = jnp.min(jnp.where(masked == m2, iota, n_experts), axis=-1,
                 keepdims=True)

    r = jnp.exp(m2 - m1)
    w1 = 1.0 / (1.0 + r)
    w1_ref[...] = w1
    w2_ref[...] = 1.0 - w1
    i1_ref[...] = i1
    i2_ref[...] = i2

    e = jnp.exp(logits - m1)
    p = e / jnp.sum(e, axis=-1, keepdims=True)
    psum = jnp.sum(p, axis=0, keepdims=True)

    @pl.when(g == 0)
    def _():
        acc_ref[...] = psum

    @pl.when(g != 0)
    def _():
        acc_ref[...] += psum

    @pl.when(g == ng - 1)
    def _():
        avg = acc_ref[...] / float(n_tokens)
        ent = -jnp.sum(avg * jnp.log(avg + 1e-8))
        max_ent = jnp.log(float(n_experts))
        dl_ref[...] = ((max_ent - ent) / max_ent).reshape(1, 1)


def kernel(x, W, b):
    B, S, H = x.shape
    E = W.shape[0]
    n_tokens = B * S
    tb = min(_TOK_BLOCK, n_tokens)
    ng = n_tokens // tb

    x2 = x.reshape(n_tokens, H)
    wt = W.T
    b2 = b.reshape(1, E)

    body = functools.partial(_routing_body, n_tokens=n_tokens, n_experts=E)
    out_shape = [
        jax.ShapeDtypeStruct((n_tokens, 1), jnp.float32),
        jax.ShapeDtypeStruct((n_tokens, 1), jnp.float32),
        jax.ShapeDtypeStruct((n_tokens, 1), jnp.int32),
        jax.ShapeDtypeStruct((n_tokens, 1), jnp.int32),
        jax.ShapeDtypeStruct((1, 1), jnp.float32),
    ]
    tok_spec = pl.BlockSpec((tb, 1), lambda g: (g, 0))
    w1, w2, i1, i2, dl = pl.pallas_call(
        body,
        grid=(ng,),
        in_specs=[
            pl.BlockSpec((tb, H), lambda g: (g, 0)),
            pl.BlockSpec((H, E), lambda g: (0, 0)),
            pl.BlockSpec((1, E), lambda g: (0, 0)),
        ],
        out_specs=[tok_spec, tok_spec, tok_spec, tok_spec,
                   pl.BlockSpec((1, 1), lambda g: (0, 0))],
        out_shape=out_shape,
        scratch_shapes=[pltpu.VMEM((1, E), jnp.float32)],
        compiler_params=pltpu.CompilerParams(
            dimension_semantics=("arbitrary",)),
    )(x2, wt, b2)

    routing_weights = jnp.concatenate([w1, w2], axis=1).reshape(B, S, 2)
    selected_experts = jnp.concatenate([i1, i2], axis=1).reshape(B, S, 2)
    return routing_weights, selected_experts, dl[0, 0]


# speedup vs baseline: 1.4024x; 1.1127x over previous
import functools

import jax
import jax.numpy as jnp
from jax import lax
from jax.experimental import pallas as pl
from jax.experimental.pallas import tpu as pltpu

_TOK_BLOCK = 1024


def _routing_body(x_ref, wt_ref, b_ref, w1_ref, w2_ref, i1_ref, i2_ref,
                  dl_ref, acc_ref, *, n_tokens, n_experts):
    g = pl.program_id(0)
    ng = pl.num_programs(0)

    logits = jnp.dot(x_ref[...], wt_ref[...],
                     preferred_element_type=jnp.float32) + b_ref[...]

    t = logits.shape[0]
    iota = lax.broadcasted_iota(jnp.int32, (t, n_experts), 1)

    m1 = jnp.max(logits, axis=-1, keepdims=True)
    i1 = jnp.min(jnp.where(logits == m1, iota, n_experts), axis=-1,
                 keepdims=True)
    masked = jnp.where(iota == i1, -jnp.inf, logits)
    m2 = jnp.max(masked, axis=-1, keepdims=True)
    i2 = jnp.min(jnp.where(masked == m2, iota, n_experts), axis=-1,
                 keepdims=True)

    r = jnp.exp(m2 - m1)
    w1 = 1.0 / (1.0 + r)
    w1_ref[...] = w1
    w2_ref[...] = 1.0 - w1
    i1_ref[...] = i1
    i2_ref[...] = i2

    e = jnp.exp(logits - m1)
    p = e / jnp.sum(e, axis=-1, keepdims=True)
    psum = jnp.sum(p, axis=0, keepdims=True)

    @pl.when(g == 0)
    def _():
        acc_ref[...] = psum

    @pl.when(g != 0)
    def _():
        acc_ref[...] += psum

    @pl.when(g == ng - 1)
    def _():
        avg = acc_ref[...] / float(n_tokens)
        ent = -jnp.sum(avg * jnp.log(avg + 1e-8))
        max_ent = jnp.log(float(n_experts))
        dl_ref[...] = ((max_ent - ent) / max_ent).reshape(1, 1)


def kernel(x, W, b):
    B, S, H = x.shape
    E = W.shape[0]
    n_tokens = B * S
    tb = min(_TOK_BLOCK, n_tokens)
    ng = n_tokens // tb

    x2 = x.reshape(n_tokens, H)
    wt = W.T
    b2 = b.reshape(1, E)

    body = functools.partial(_routing_body, n_tokens=n_tokens, n_experts=E)
    out_shape = [
        jax.ShapeDtypeStruct((n_tokens, 1), jnp.float32),
        jax.ShapeDtypeStruct((n_tokens, 1), jnp.float32),
        jax.ShapeDtypeStruct((n_tokens, 1), jnp.int32),
        jax.ShapeDtypeStruct((n_tokens, 1), jnp.int32),
        jax.ShapeDtypeStruct((1, 1), jnp.float32),
    ]
    tok_spec = pl.BlockSpec((tb, 1), lambda g: (g, 0))
    w1, w2, i1, i2, dl = pl.pallas_call(
        body,
        grid=(ng,),
        in_specs=[
            pl.BlockSpec((tb, H), lambda g: (g, 0)),
            pl.BlockSpec((H, E), lambda g: (0, 0)),
            pl.BlockSpec((1, E), lambda g: (0, 0)),
        ],
        out_specs=[tok_spec, tok_spec, tok_spec, tok_spec,
                   pl.BlockSpec((1, 1), lambda g: (0, 0))],
        out_shape=out_shape,
        scratch_shapes=[pltpu.VMEM((1, E), jnp.float32)],
        compiler_params=pltpu.CompilerParams(
            dimension_semantics=("arbitrary",)),
    )(x2, wt, b2)

    routing_weights = jnp.concatenate([w1, w2], axis=1).reshape(B, S, 2)
    selected_experts = jnp.concatenate([i1, i2], axis=1).reshape(B, S, 2)
    return routing_weights, selected_experts, dl[0, 0]


# speedup vs baseline: 1.4599x; 1.0410x over previous
import functools

import jax
import jax.numpy as jnp
from jax import lax
from jax.experimental import pallas as pl
from jax.experimental.pallas import tpu as pltpu

_TOK_BLOCK = 2048


def _routing_body(x_ref, wt_ref, b_ref, w1_ref, w2_ref, i1_ref, i2_ref,
                  dl_ref, acc_ref, *, n_tokens, n_experts):
    g = pl.program_id(0)
    ng = pl.num_programs(0)

    logits = jnp.dot(x_ref[...], wt_ref[...],
                     preferred_element_type=jnp.float32) + b_ref[...]

    t = logits.shape[0]
    iota = lax.broadcasted_iota(jnp.int32, (t, n_experts), 1)

    m1 = jnp.max(logits, axis=-1, keepdims=True)
    i1 = jnp.min(jnp.where(logits == m1, iota, n_experts), axis=-1,
                 keepdims=True)
    masked = jnp.where(iota == i1, -jnp.inf, logits)
    m2 = jnp.max(masked, axis=-1, keepdims=True)
    i2 = jnp.min(jnp.where(masked == m2, iota, n_experts), axis=-1,
                 keepdims=True)

    r = jnp.exp(m2 - m1)
    w1 = 1.0 / (1.0 + r)
    w1_ref[...] = w1
    w2_ref[...] = 1.0 - w1
    i1_ref[...] = i1
    i2_ref[...] = i2

    e = jnp.exp(logits - m1)
    p = e / jnp.sum(e, axis=-1, keepdims=True)
    psum = jnp.sum(p, axis=0, keepdims=True)

    @pl.when(g == 0)
    def _():
        acc_ref[...] = psum

    @pl.when(g != 0)
    def _():
        acc_ref[...] += psum

    @pl.when(g == ng - 1)
    def _():
        avg = acc_ref[...] / float(n_tokens)
        ent = -jnp.sum(avg * jnp.log(avg + 1e-8))
        max_ent = jnp.log(float(n_experts))
        dl_ref[...] = ((max_ent - ent) / max_ent).reshape(1, 1)


def kernel(x, W, b):
    B, S, H = x.shape
    E = W.shape[0]
    n_tokens = B * S
    tb = min(_TOK_BLOCK, n_tokens)
    ng = n_tokens // tb

    x2 = x.reshape(n_tokens, H)
    wt = W.T
    b2 = b.reshape(1, E)

    body = functools.partial(_routing_body, n_tokens=n_tokens, n_experts=E)
    out_shape = [
        jax.ShapeDtypeStruct((n_tokens, 1), jnp.float32),
        jax.ShapeDtypeStruct((n_tokens, 1), jnp.float32),
        jax.ShapeDtypeStruct((n_tokens, 1), jnp.int32),
        jax.ShapeDtypeStruct((n_tokens, 1), jnp.int32),
        jax.ShapeDtypeStruct((1, 1), jnp.float32),
    ]
    tok_spec = pl.BlockSpec((tb, 1), lambda g: (g, 0))
    w1, w2, i1, i2, dl = pl.pallas_call(
        body,
        grid=(ng,),
        in_specs=[
            pl.BlockSpec((tb, H), lambda g: (g, 0)),
            pl.BlockSpec((H, E), lambda g: (0, 0)),
            pl.BlockSpec((1, E), lambda g: (0, 0)),
        ],
        out_specs=[tok_spec, tok_spec, tok_spec, tok_spec,
                   pl.BlockSpec((1, 1), lambda g: (0, 0))],
        out_shape=out_shape,
        scratch_shapes=[pltpu.VMEM((1, E), jnp.float32)],
        compiler_params=pltpu.CompilerParams(
            dimension_semantics=("arbitrary",)),
    )(x2, wt, b2)

    routing_weights = jnp.concatenate([w1, w2], axis=1).reshape(B, S, 2)
    selected_experts = jnp.concatenate([i1, i2], axis=1).reshape(B, S, 2)
    return routing_weights, selected_experts, dl[0, 0]
